# baseline (speedup 1.0000x reference)
import functools

import jax
import jax.numpy as jnp
from jax import lax
from jax.experimental import pallas as pl
from jax.experimental.pallas import tpu as pltpu
from jax.experimental.pallas import tpu_sc as plsc

HID = 128
NMOL = 4096
STEPS = 6

NC, NS, L = 2, 16, 16
NW = NC * NS
N_PAD = 102400
APT = N_PAD // NW
BLK = 80
NBLK = APT // BLK
NGRP = BLK // L
ACC_ROWS = 4352
STRIPE = ACC_ROWS // NS
H_PAD_ROWS = 4104

_sc_mesh = plsc.VectorSubcoreMesh(
    core_axis_name="c", subcore_axis_name="s", num_cores=NC, num_subcores=NS)


def _attn_body(x_hbm, seg_hbm, h_hbm, num_hbm, den_hbm, *sc):
    seg_vs = sc[0:4]
    x_vs = sc[4:8]
    h_vs = sc[8:12]
    zv, den_v, bf_v, acc = sc[12:16]
    sem_hs = sc[16:20]
    sem_xs = sc[20:24]
    sem_ss = sc[24:28]
    h_v0 = h_vs[0]
    c = lax.axis_index("c")
    s = lax.axis_index("s")

    zero16 = jnp.zeros((L,), jnp.float32)

    def zrow(i, _):
        for k in range(HID // L):
            h_v0[i, pl.ds(L * k, L)] = zero16
        return 0
    lax.fori_loop(0, BLK, zrow, 0)

    row0 = s * STRIPE
    for t in range(STRIPE // BLK):
        pltpu.sync_copy(h_v0, acc.at[pl.ds(row0 + t * BLK, BLK)])
    rem = STRIPE - (STRIPE // BLK) * BLK
    if rem:
        pltpu.sync_copy(h_v0.at[pl.ds(0, rem)],
                        acc.at[pl.ds(row0 + (STRIPE // BLK) * BLK, rem)])

    def zden(j, _):
        den_v[pl.ds(L * j, L)] = zero16
        return 0
    lax.fori_loop(0, ACC_ROWS // L, zden, 0)
    plsc.subcore_barrier()

    wid = s * NC + c
    base = wid * APT
    lanes = lax.iota(jnp.int32, L)
    onehots = [(lanes == j).astype(jnp.float32) for j in range(L)]
    rowids = [jnp.full((L,), j, jnp.int32) for j in range(L)]

    def compute(r):
        seg_v, x_v, h_v = seg_vs[r], x_vs[r], h_vs[r]

        def grp(g, _):
            seg16 = seg_v[pl.ds(g * L, L)]
            wlp = zero16
            for j in range(L):
                a = g * L + j
                ps = []
                for k in range(HID // L):
                    ps.append(x_v[a, pl.ds(L * k, L)] * h_v[a, pl.ds(L * k, L)])
                while len(ps) > 1:
                    ps = [ps[i] + ps[i + 1] for i in range(0, len(ps), 2)]
                v = ps[0]
                for m in (8, 4, 2, 1):
                    bf_v[j, pl.ds(0, L)] = v
                    v = v + plsc.load_gather(bf_v, [rowids[j], lanes ^ m])
                w16 = jnp.exp(v)
                for k in range(HID // L):
                    x_v[a, pl.ds(L * k, L)] = w16 * x_v[a, pl.ds(L * k, L)]
                wlp = wlp + w16 * onehots[j]
            g16 = g * L
            wacc = wlp
            for d in (1, 2, 4, 8):
                zv[1, pl.ds(0, L)] = wacc
                idx = jnp.minimum(lanes + d, L - 1)
                s_dn = plsc.load_gather(zv, [rowids[1], idx])
                seg_dn = plsc.load_gather(seg_v, [g16 + idx])
                ok = (lanes + d <= L - 1) & (seg_dn == seg16)
                wacc = wacc + jnp.where(ok, s_dn, 0.0)
            prev = plsc.load_gather(
                seg_v, [g16 + jnp.maximum(lanes - 1, 0)])
            start = (lanes == 0) | (seg16 != prev)
            plsc.addupdate_scatter(
                den_v, [seg16], jnp.where(start, wacc, 0.0))
            return 0
        lax.fori_loop(0, NGRP, grp, 0)

    def quad_body(i, _):
        offs = [base + (4 * i + u) * BLK for u in range(4)]
        for u in range(4):
            pltpu.sync_copy(seg_hbm.at[pl.ds(offs[u], BLK)], seg_vs[u])
        ags, axs = [], []
        for u in range(4):
            ags.append(pltpu.async_copy(h_hbm.at[seg_vs[u]], h_vs[u],
                                        sem_hs[u]))
            axs.append(pltpu.async_copy(x_hbm.at[pl.ds(offs[u], BLK)],
                                        x_vs[u], sem_xs[u]))
        ass = []
        for u in range(4):
            ags[u].wait()
            axs[u].wait()
            compute(u)
            ass.append(pltpu.async_copy(x_vs[u], acc.at[seg_vs[u]],
                                        sem_ss[u], add=True))
        for u in range(4):
            ass[u].wait()
        return 0
    lax.fori_loop(0, NBLK // 4, quad_body, 0)
    plsc.subcore_barrier()

    pltpu.sync_copy(acc.at[pl.ds(row0, STRIPE)],
                    num_hbm.at[pl.ds(c * ACC_ROWS + row0, STRIPE)])
    pltpu.sync_copy(den_v, den_hbm.at[wid])


_attn = functools.partial(
    pl.kernel,
    out_type=(
        jax.ShapeDtypeStruct((NC * ACC_ROWS, HID), jnp.float32),
        jax.ShapeDtypeStruct((NW, ACC_ROWS), jnp.float32),
    ),
    mesh=_sc_mesh,
    compiler_params=pltpu.CompilerParams(
        needs_layout_passes=False, disable_bounds_checks=True),
    scratch_types=(
        [pltpu.VMEM((BLK,), jnp.int32) for _ in range(4)]
        + [pltpu.VMEM((BLK, HID), jnp.float32) for _ in range(4)]
        + [pltpu.VMEM((BLK, HID), jnp.float32) for _ in range(4)]
        + [
            pltpu.VMEM((L, HID), jnp.float32),
            pltpu.VMEM((ACC_ROWS,), jnp.float32),
            pltpu.VMEM((L, L), jnp.float32),
            pltpu.VMEM_SHARED((ACC_ROWS, HID), jnp.float32),
        ]
        + [pltpu.SemaphoreType.DMA for _ in range(12)]
    ),
)(_attn_body)


def _lstm_body(h_ref, c_ref, num_ref, den_ref, u_ref, b_ref, q_ref, h_out, c_out):
    num = num_ref[0] + num_ref[1]
    den = jnp.sum(den_ref[...], axis=0)
    rinv = jnp.where(den > 0, 1.0 / den, 0.0)
    r = num * rinv[:, None]
    h = h_ref[...]
    q = jnp.concatenate([h, r], axis=1)
    q_ref[...] = q
    z = jnp.dot(q, u_ref[...], preferred_element_type=jnp.float32) + b_ref[...]
    i = jax.nn.sigmoid(z[:, :HID])
    f = jax.nn.sigmoid(z[:, HID:2 * HID])
    o = jax.nn.sigmoid(z[:, 2 * HID:3 * HID])
    g = z[:, 3 * HID:]
    c_new = f * c_ref[...] + i * jnp.tanh(g)
    h_out[...] = o * jnp.tanh(c_new)
    c_out[...] = c_new


_ROWS_BLK = 256
_lstm = pl.pallas_call(
    _lstm_body,
    grid=(NMOL // _ROWS_BLK,),
    in_specs=[
        pl.BlockSpec((_ROWS_BLK, HID), lambda i: (i, 0)),
        pl.BlockSpec((_ROWS_BLK, HID), lambda i: (i, 0)),
        pl.BlockSpec((2, _ROWS_BLK, HID), lambda i: (0, i, 0)),
        pl.BlockSpec((NW, _ROWS_BLK), lambda i: (0, i)),
        pl.BlockSpec((2 * HID, 4 * HID), lambda i: (0, 0)),
        pl.BlockSpec((1, 4 * HID), lambda i: (0, 0)),
    ],
    out_specs=[
        pl.BlockSpec((_ROWS_BLK, 2 * HID), lambda i: (i, 0)),
        pl.BlockSpec((_ROWS_BLK, HID), lambda i: (i, 0)),
        pl.BlockSpec((_ROWS_BLK, HID), lambda i: (i, 0)),
    ],
    out_shape=[
        jax.ShapeDtypeStruct((NMOL, 2 * HID), jnp.float32),
        jax.ShapeDtypeStruct((NMOL, HID), jnp.float32),
        jax.ShapeDtypeStruct((NMOL, HID), jnp.float32),
    ],
)


def kernel(atom_features, atom_split, U, b):
    n = atom_features.shape[0]
    seg = atom_split.astype(jnp.int32)
    xp = jnp.concatenate(
        [atom_features, jnp.zeros((N_PAD - n, HID), jnp.float32)], axis=0)
    segp = jnp.concatenate(
        [seg, jnp.full((N_PAD - n,), NMOL, jnp.int32)], axis=0)
    b2 = b.reshape(1, 4 * HID)

    h = jnp.zeros((NMOL, HID), jnp.float32)
    c = jnp.zeros((NMOL, HID), jnp.float32)
    q0 = jnp.zeros((NMOL, 2 * HID), jnp.float32)

    def step(_, carry):
        h, c, _q = carry
        hp = jnp.concatenate(
            [h, jnp.zeros((H_PAD_ROWS - NMOL, HID), jnp.float32)], axis=0)
        num, den = _attn(xp, segp, hp)
        nump = num.reshape(NC, ACC_ROWS, HID)[:, :NMOL, :]
        denp = den[:, :NMOL]
        q, h, c = _lstm(h, c, nump, denp, U, b2)
        return h, c, q

    _, _, q = lax.fori_loop(0, STEPS, step, (h, c, q0))
    return q


# speedup vs baseline: 1.1498x; 1.1498x over previous
import functools

import jax
import jax.numpy as jnp
from jax import lax
from jax.experimental import pallas as pl
from jax.experimental.pallas import tpu as pltpu
from jax.experimental.pallas import tpu_sc as plsc

HID = 128
NMOL = 4096
STEPS = 6

NC, NS, L = 2, 16, 16
NW = NC * NS
N_PAD = 100352
APT = N_PAD // NW
BLK = 112
NBLK = APT // BLK
NGRP = BLK // L
ACC_ROWS = 4352
STRIPE = ACC_ROWS // NS
H_PAD_ROWS = 4104

_sc_mesh = plsc.VectorSubcoreMesh(
    core_axis_name="c", subcore_axis_name="s", num_cores=NC, num_subcores=NS)


def _attn_body(x_hbm, seg_hbm, h_hbm, num_hbm, den_hbm,
               seg_v0, seg_v1, x_v0, x_v1, h_v0, h_v1,
               zv, den_v, bf_v, acc,
               sem_h0, sem_h1, sem_x0, sem_x1, sem_s0, sem_s1):
    seg_vs = (seg_v0, seg_v1)
    x_vs = (x_v0, x_v1)
    h_vs = (h_v0, h_v1)
    c = lax.axis_index("c")
    s = lax.axis_index("s")

    zero16 = jnp.zeros((L,), jnp.float32)

    def zrow(i, _):
        for k in range(HID // L):
            h_v0[i, pl.ds(L * k, L)] = zero16
        return 0
    lax.fori_loop(0, BLK, zrow, 0)

    row0 = s * STRIPE
    pltpu.sync_copy(h_v0, acc.at[pl.ds(row0, BLK)])
    pltpu.sync_copy(h_v0, acc.at[pl.ds(row0 + BLK, BLK)])
    pltpu.sync_copy(h_v0.at[pl.ds(0, STRIPE - 2 * BLK)],
                    acc.at[pl.ds(row0 + 2 * BLK, STRIPE - 2 * BLK)])

    def zden(j, _):
        den_v[pl.ds(L * j, L)] = zero16
        return 0
    lax.fori_loop(0, ACC_ROWS // L, zden, 0)
    plsc.subcore_barrier()

    wid = s * NC + c
    base = wid * APT
    lanes = lax.iota(jnp.int32, L)
    onehots = [(lanes == j).astype(jnp.float32) for j in range(L)]
    rowids = [jnp.full((L,), j, jnp.int32) for j in range(L)]

    def compute(r):
        seg_v, x_v, h_v = seg_vs[r], x_vs[r], h_vs[r]

        def grp(g, _):
            seg16 = seg_v[pl.ds(g * L, L)]
            wlp = zero16
            for j in range(L):
                a = g * L + j
                ps = []
                for k in range(HID // L):
                    ps.append(x_v[a, pl.ds(L * k, L)] * h_v[a, pl.ds(L * k, L)])
                while len(ps) > 1:
                    ps = [ps[i] + ps[i + 1] for i in range(0, len(ps), 2)]
                v = ps[0]
                for m in (8, 4, 2, 1):
                    bf_v[j, pl.ds(0, L)] = v
                    v = v + plsc.load_gather(bf_v, [rowids[j], lanes ^ m])
                w16 = jnp.exp(v)
                for k in range(HID // L):
                    x_v[a, pl.ds(L * k, L)] = w16 * x_v[a, pl.ds(L * k, L)]
                wlp = wlp + w16 * onehots[j]
            g16 = g * L
            wacc = wlp
            for d in (1, 2, 4, 8):
                zv[1, pl.ds(0, L)] = wacc
                idx = jnp.minimum(lanes + d, L - 1)
                s_dn = plsc.load_gather(zv, [rowids[1], idx])
                seg_dn = plsc.load_gather(seg_v, [g16 + idx])
                ok = (lanes + d <= L - 1) & (seg_dn == seg16)
                wacc = wacc + jnp.where(ok, s_dn, 0.0)
            prev = plsc.load_gather(
                seg_v, [g16 + jnp.maximum(lanes - 1, 0)])
            start = (lanes == 0) | (seg16 != prev)
            plsc.addupdate_scatter(
                den_v, [seg16], jnp.where(start, wacc, 0.0))
            return 0
        lax.fori_loop(0, NGRP, grp, 0)

    def pair_body(i, _):
        off0 = base + (2 * i) * BLK
        off1 = off0 + BLK
        pltpu.sync_copy(seg_hbm.at[pl.ds(off0, BLK)], seg_v0)
        pltpu.sync_copy(seg_hbm.at[pl.ds(off1, BLK)], seg_v1)
        ag0 = pltpu.async_copy(h_hbm.at[seg_v0], h_v0, sem_h0)
        ax0 = pltpu.async_copy(x_hbm.at[pl.ds(off0, BLK)], x_v0, sem_x0)
        ag1 = pltpu.async_copy(h_hbm.at[seg_v1], h_v1, sem_h1)
        ax1 = pltpu.async_copy(x_hbm.at[pl.ds(off1, BLK)], x_v1, sem_x1)
        ag0.wait()
        ax0.wait()
        compute(0)
        as0 = pltpu.async_copy(x_v0, acc.at[seg_v0], sem_s0, add=True)
        ag1.wait()
        ax1.wait()
        compute(1)
        as1 = pltpu.async_copy(x_v1, acc.at[seg_v1], sem_s1, add=True)
        as0.wait()
        as1.wait()
        return 0
    lax.fori_loop(0, NBLK // 2, pair_body, 0)
    plsc.subcore_barrier()

    pltpu.sync_copy(acc.at[pl.ds(row0, STRIPE)],
                    num_hbm.at[pl.ds(c * ACC_ROWS + row0, STRIPE)])
    pltpu.sync_copy(den_v, den_hbm.at[wid])


_attn = functools.partial(
    pl.kernel,
    out_type=(
        jax.ShapeDtypeStruct((NC * ACC_ROWS, HID), jnp.float32),
        jax.ShapeDtypeStruct((NW, ACC_ROWS), jnp.float32),
    ),
    mesh=_sc_mesh,
    compiler_params=pltpu.CompilerParams(
        needs_layout_passes=False, disable_bounds_checks=True),
    scratch_types=(
        [pltpu.VMEM((BLK,), jnp.int32) for _ in range(2)]
        + [pltpu.VMEM((BLK, HID), jnp.float32) for _ in range(2)]
        + [pltpu.VMEM((BLK, HID), jnp.float32) for _ in range(2)]
        + [
            pltpu.VMEM((L, HID), jnp.float32),
            pltpu.VMEM((ACC_ROWS,), jnp.float32),
            pltpu.VMEM((L, L), jnp.float32),
            pltpu.VMEM_SHARED((ACC_ROWS, HID), jnp.float32),
        ]
        + [pltpu.SemaphoreType.DMA for _ in range(6)]
    ),
)(_attn_body)


def _lstm_body(h_ref, c_ref, num_ref, den_ref, u_ref, b_ref, q_ref, h_out, c_out):
    num = num_ref[0] + num_ref[1]
    den = jnp.sum(den_ref[...], axis=0)
    rinv = jnp.where(den > 0, 1.0 / den, 0.0)
    r = num * rinv[:, None]
    h = h_ref[...]
    q = jnp.concatenate([h, r], axis=1)
    q_ref[...] = q
    z = jnp.dot(q, u_ref[...], preferred_element_type=jnp.float32) + b_ref[...]
    i = jax.nn.sigmoid(z[:, :HID])
    f = jax.nn.sigmoid(z[:, HID:2 * HID])
    o = jax.nn.sigmoid(z[:, 2 * HID:3 * HID])
    g = z[:, 3 * HID:]
    c_new = f * c_ref[...] + i * jnp.tanh(g)
    h_out[...] = o * jnp.tanh(c_new)
    c_out[...] = c_new


_ROWS_BLK = 256
_lstm = pl.pallas_call(
    _lstm_body,
    grid=(NMOL // _ROWS_BLK,),
    in_specs=[
        pl.BlockSpec((_ROWS_BLK, HID), lambda i: (i, 0)),
        pl.BlockSpec((_ROWS_BLK, HID), lambda i: (i, 0)),
        pl.BlockSpec((2, _ROWS_BLK, HID), lambda i: (0, i, 0)),
        pl.BlockSpec((NW, _ROWS_BLK), lambda i: (0, i)),
        pl.BlockSpec((2 * HID, 4 * HID), lambda i: (0, 0)),
        pl.BlockSpec((1, 4 * HID), lambda i: (0, 0)),
    ],
    out_specs=[
        pl.BlockSpec((_ROWS_BLK, 2 * HID), lambda i: (i, 0)),
        pl.BlockSpec((_ROWS_BLK, HID), lambda i: (i, 0)),
        pl.BlockSpec((_ROWS_BLK, HID), lambda i: (i, 0)),
    ],
    out_shape=[
        jax.ShapeDtypeStruct((NMOL, 2 * HID), jnp.float32),
        jax.ShapeDtypeStruct((NMOL, HID), jnp.float32),
        jax.ShapeDtypeStruct((NMOL, HID), jnp.float32),
    ],
)


def kernel(atom_features, atom_split, U, b):
    n = atom_features.shape[0]
    seg = atom_split.astype(jnp.int32)
    xp = jnp.concatenate(
        [atom_features, jnp.zeros((N_PAD - n, HID), jnp.float32)], axis=0)
    segp = jnp.concatenate(
        [seg, jnp.full((N_PAD - n,), NMOL, jnp.int32)], axis=0)
    b2 = b.reshape(1, 4 * HID)

    h = jnp.zeros((NMOL, HID), jnp.float32)
    c = jnp.zeros((NMOL, HID), jnp.float32)
    q0 = jnp.zeros((NMOL, 2 * HID), jnp.float32)

    def step(_, carry):
        h, c, _q = carry
        hp = jnp.concatenate(
            [h, jnp.zeros((H_PAD_ROWS - NMOL, HID), jnp.float32)], axis=0)
        num, den = _attn(xp, segp, hp)
        nump = num.reshape(NC, ACC_ROWS, HID)[:, :NMOL, :]
        denp = den[:, :NMOL]
        q, h, c = _lstm(h, c, nump, denp, U, b2)
        return h, c, q

    _, _, q = lax.fori_loop(0, STEPS, step, (h, c, q0))
    return q


# speedup vs baseline: 1.1551x; 1.0046x over previous
import functools

import jax
import jax.numpy as jnp
from jax import lax
from jax.experimental import pallas as pl
from jax.experimental.pallas import tpu as pltpu
from jax.experimental.pallas import tpu_sc as plsc

HID = 128
NMOL = 4096
STEPS = 6

NC, NS, L = 2, 16, 16
NW = NC * NS
N_PAD = 100352
APT = N_PAD // NW
BLK = 112
NBLK = APT // BLK
NGRP = BLK // L
ACC_ROWS = 4352
STRIPE = ACC_ROWS // NS
H_PAD_ROWS = 4104

_sc_mesh = plsc.VectorSubcoreMesh(
    core_axis_name="c", subcore_axis_name="s", num_cores=NC, num_subcores=NS)


def _attn_body(x_hbm, seg_hbm, h_hbm, num_hbm, den_hbm,
               seg_v0, seg_v1, x_v0, x_v1, h_v0, h_v1,
               zv, den_v, bf_v, acc,
               sem_h0, sem_h1, sem_x0, sem_x1, sem_s0, sem_s1):
    seg_vs = (seg_v0, seg_v1)
    x_vs = (x_v0, x_v1)
    h_vs = (h_v0, h_v1)
    c = lax.axis_index("c")
    s = lax.axis_index("s")

    zero16 = jnp.zeros((L,), jnp.float32)

    def zrow(i, _):
        for k in range(HID // L):
            h_v0[i, pl.ds(L * k, L)] = zero16
        return 0
    lax.fori_loop(0, BLK, zrow, 0)

    row0 = s * STRIPE
    pltpu.sync_copy(h_v0, acc.at[pl.ds(row0, BLK)])
    pltpu.sync_copy(h_v0, acc.at[pl.ds(row0 + BLK, BLK)])
    pltpu.sync_copy(h_v0.at[pl.ds(0, STRIPE - 2 * BLK)],
                    acc.at[pl.ds(row0 + 2 * BLK, STRIPE - 2 * BLK)])

    def zden(j, _):
        den_v[pl.ds(L * j, L)] = zero16
        return 0
    lax.fori_loop(0, ACC_ROWS // L, zden, 0)
    plsc.subcore_barrier()

    wid = s * NC + c
    base = wid * APT
    lanes = lax.iota(jnp.int32, L)
    onehots = [(lanes == j).astype(jnp.float32) for j in range(L)]
    rowids = [jnp.full((L,), j, jnp.int32) for j in range(L)]

    def compute(r):
        seg_v, x_v, h_v = seg_vs[r], x_vs[r], h_vs[r]

        def grp(g, _):
            seg16 = seg_v[pl.ds(g * L, L)]
            wlp = zero16
            for j in range(L):
                a = g * L + j
                acc16 = x_v[a, pl.ds(0, L)] * h_v[a, pl.ds(0, L)]
                for k in range(1, HID // L):
                    acc16 = acc16 + x_v[a, pl.ds(L * k, L)] * h_v[a, pl.ds(L * k, L)]
                v = acc16
                for m in (8, 4, 2, 1):
                    bf_v[j, pl.ds(0, L)] = v
                    v = v + plsc.load_gather(bf_v, [rowids[j], lanes ^ m])
                w16 = jnp.exp(v)
                for k in range(HID // L):
                    x_v[a, pl.ds(L * k, L)] = w16 * x_v[a, pl.ds(L * k, L)]
                wlp = wlp + w16 * onehots[j]
            g16 = g * L
            wacc = wlp
            for d in (1, 2, 4, 8):
                zv[1, pl.ds(0, L)] = wacc
                idx = jnp.minimum(lanes + d, L - 1)
                s_dn = plsc.load_gather(zv, [rowids[1], idx])
                seg_dn = plsc.load_gather(seg_v, [g16 + idx])
                ok = (lanes + d <= L - 1) & (seg_dn == seg16)
                wacc = wacc + jnp.where(ok, s_dn, 0.0)
            prev = plsc.load_gather(
                seg_v, [g16 + jnp.maximum(lanes - 1, 0)])
            start = (lanes == 0) | (seg16 != prev)
            plsc.addupdate_scatter(
                den_v, [seg16], jnp.where(start, wacc, 0.0))
            return 0
        lax.fori_loop(0, NGRP, grp, 0)

    def pair_body(i, _):
        off0 = base + (2 * i) * BLK
        off1 = off0 + BLK
        pltpu.sync_copy(seg_hbm.at[pl.ds(off0, BLK)], seg_v0)
        pltpu.sync_copy(seg_hbm.at[pl.ds(off1, BLK)], seg_v1)
        ag0 = pltpu.async_copy(h_hbm.at[seg_v0], h_v0, sem_h0)
        ax0 = pltpu.async_copy(x_hbm.at[pl.ds(off0, BLK)], x_v0, sem_x0)
        ag1 = pltpu.async_copy(h_hbm.at[seg_v1], h_v1, sem_h1)
        ax1 = pltpu.async_copy(x_hbm.at[pl.ds(off1, BLK)], x_v1, sem_x1)
        ag0.wait()
        ax0.wait()
        compute(0)
        as0 = pltpu.async_copy(x_v0, acc.at[seg_v0], sem_s0, add=True)
        ag1.wait()
        ax1.wait()
        compute(1)
        as1 = pltpu.async_copy(x_v1, acc.at[seg_v1], sem_s1, add=True)
        as0.wait()
        as1.wait()
        return 0
    lax.fori_loop(0, NBLK // 2, pair_body, 0)
    plsc.subcore_barrier()

    pltpu.sync_copy(acc.at[pl.ds(row0, STRIPE)],
                    num_hbm.at[pl.ds(c * ACC_ROWS + row0, STRIPE)])
    pltpu.sync_copy(den_v, den_hbm.at[wid])


_attn = functools.partial(
    pl.kernel,
    out_type=(
        jax.ShapeDtypeStruct((NC * ACC_ROWS, HID), jnp.float32),
        jax.ShapeDtypeStruct((NW, ACC_ROWS), jnp.float32),
    ),
    mesh=_sc_mesh,
    compiler_params=pltpu.CompilerParams(
        needs_layout_passes=False, disable_bounds_checks=True),
    scratch_types=(
        [pltpu.VMEM((BLK,), jnp.int32) for _ in range(2)]
        + [pltpu.VMEM((BLK, HID), jnp.float32) for _ in range(2)]
        + [pltpu.VMEM((BLK, HID), jnp.float32) for _ in range(2)]
        + [
            pltpu.VMEM((L, HID), jnp.float32),
            pltpu.VMEM((ACC_ROWS,), jnp.float32),
            pltpu.VMEM((L, L), jnp.float32),
            pltpu.VMEM_SHARED((ACC_ROWS, HID), jnp.float32),
        ]
        + [pltpu.SemaphoreType.DMA for _ in range(6)]
    ),
)(_attn_body)


def _lstm_body(h_ref, c_ref, num_ref, den_ref, u_ref, b_ref, q_ref, h_out, c_out):
    num = num_ref[0] + num_ref[1]
    den = jnp.sum(den_ref[...], axis=0)
    rinv = jnp.where(den > 0, 1.0 / den, 0.0)
    r = num * rinv[:, None]
    h = h_ref[...]
    q = jnp.concatenate([h, r], axis=1)
    q_ref[...] = q
    z = jnp.dot(q, u_ref[...], preferred_element_type=jnp.float32) + b_ref[...]
    i = jax.nn.sigmoid(z[:, :HID])
    f = jax.nn.sigmoid(z[:, HID:2 * HID])
    o = jax.nn.sigmoid(z[:, 2 * HID:3 * HID])
    g = z[:, 3 * HID:]
    c_new = f * c_ref[...] + i * jnp.tanh(g)
    h_out[...] = o * jnp.tanh(c_new)
    c_out[...] = c_new


_ROWS_BLK = 256
_lstm = pl.pallas_call(
    _lstm_body,
    grid=(NMOL // _ROWS_BLK,),
    in_specs=[
        pl.BlockSpec((_ROWS_BLK, HID), lambda i: (i, 0)),
        pl.BlockSpec((_ROWS_BLK, HID), lambda i: (i, 0)),
        pl.BlockSpec((2, _ROWS_BLK, HID), lambda i: (0, i, 0)),
        pl.BlockSpec((NW, _ROWS_BLK), lambda i: (0, i)),
        pl.BlockSpec((2 * HID, 4 * HID), lambda i: (0, 0)),
        pl.BlockSpec((1, 4 * HID), lambda i: (0, 0)),
    ],
    out_specs=[
        pl.BlockSpec((_ROWS_BLK, 2 * HID), lambda i: (i, 0)),
        pl.BlockSpec((_ROWS_BLK, HID), lambda i: (i, 0)),
        pl.BlockSpec((_ROWS_BLK, HID), lambda i: (i, 0)),
    ],
    out_shape=[
        jax.ShapeDtypeStruct((NMOL, 2 * HID), jnp.float32),
        jax.ShapeDtypeStruct((NMOL, HID), jnp.float32),
        jax.ShapeDtypeStruct((NMOL, HID), jnp.float32),
    ],
)


def kernel(atom_features, atom_split, U, b):
    n = atom_features.shape[0]
    seg = atom_split.astype(jnp.int32)
    xp = jnp.concatenate(
        [atom_features, jnp.zeros((N_PAD - n, HID), jnp.float32)], axis=0)
    segp = jnp.concatenate(
        [seg, jnp.full((N_PAD - n,), NMOL, jnp.int32)], axis=0)
    b2 = b.reshape(1, 4 * HID)

    h = jnp.zeros((NMOL, HID), jnp.float32)
    c = jnp.zeros((NMOL, HID), jnp.float32)
    q0 = jnp.zeros((NMOL, 2 * HID), jnp.float32)

    def step(_, carry):
        h, c, _q = carry
        hp = jnp.concatenate(
            [h, jnp.zeros((H_PAD_ROWS - NMOL, HID), jnp.float32)], axis=0)
        num, den = _attn(xp, segp, hp)
        nump = num.reshape(NC, ACC_ROWS, HID)[:, :NMOL, :]
        denp = den[:, :NMOL]
        q, h, c = _lstm(h, c, nump, denp, U, b2)
        return h, c, q

    _, _, q = lax.fori_loop(0, STEPS, step, (h, c, q0))
    return q
